# baseline (speedup 1.0000x reference)
import functools

import jax
import jax.numpy as jnp
from jax.experimental import pallas as pl
from jax.experimental.pallas import tpu as pltpu

EMBED_DIM = 768
HIDDEN_DIM = 3072
NUM_EXPERTS = 8
TOP_K = 2

BH = 768
NHB = HIDDEN_DIM // BH


def _moe_dense_kernel(x_ref, wr_ref, br_ref, w1_ref, b1_ref, w2_ref, b2_ref,
                      out_ref):
    e = pl.program_id(0)
    hb = pl.program_id(1)

    xb = x_ref[...]

    logits = jnp.dot(xb, wr_ref[...], preferred_element_type=jnp.float32)
    logits = logits + br_ref[...]
    lane = jax.lax.broadcasted_iota(jnp.int32, logits.shape, 1)
    m1 = jnp.max(logits, axis=1, keepdims=True)
    i1 = jnp.min(jnp.where(logits == m1, lane, NUM_EXPERTS), axis=1,
                 keepdims=True)
    l2 = jnp.where(lane == i1, -jnp.inf, logits)
    m2 = jnp.max(l2, axis=1, keepdims=True)
    i2 = jnp.min(jnp.where(l2 == m2, lane, NUM_EXPERTS), axis=1, keepdims=True)
    p2 = 1.0 / (1.0 + jnp.exp(m1 - m2))
    p1 = 1.0 - p2
    w_e = jnp.where(i1 == e, p1, 0.0) + jnp.where(i2 == e, p2, 0.0)

    @pl.when((e == 0) & (hb == 0))
    def _():
        out_ref[...] = jnp.zeros_like(out_ref)

    h = jnp.dot(xb, w1_ref[0], preferred_element_type=jnp.float32)
    h = jnp.maximum(h + b1_ref[0], 0.0)
    part = jnp.dot(h, w2_ref[0], preferred_element_type=jnp.float32)

    @pl.when(hb == 0)
    def _():
        out_ref[...] += w_e * b2_ref[0]

    out_ref[...] += w_e * part


def kernel(x, Wr, br, W1, b1, W2, b2):
    batch, seq, d = x.shape
    x_flat = x.reshape(-1, d)
    T = x_flat.shape[0]

    out = pl.pallas_call(
        _moe_dense_kernel,
        grid=(NUM_EXPERTS, NHB),
        in_specs=[
            pl.BlockSpec((T, d), lambda e, hb: (0, 0)),
            pl.BlockSpec((d, NUM_EXPERTS), lambda e, hb: (0, 0)),
            pl.BlockSpec((1, NUM_EXPERTS), lambda e, hb: (0, 0)),
            pl.BlockSpec((1, d, BH), lambda e, hb: (e, 0, hb)),
            pl.BlockSpec((1, 1, BH), lambda e, hb: (e, 0, hb)),
            pl.BlockSpec((1, BH, d), lambda e, hb: (e, hb, 0)),
            pl.BlockSpec((1, 1, d), lambda e, hb: (e, 0, 0)),
        ],
        out_specs=pl.BlockSpec((T, d), lambda e, hb: (0, 0)),
        out_shape=jax.ShapeDtypeStruct((T, d), jnp.float32),
    )(x_flat, Wr, br.reshape(1, NUM_EXPERTS),
      W1, b1.reshape(NUM_EXPERTS, 1, HIDDEN_DIM),
      W2, b2.reshape(NUM_EXPERTS, 1, EMBED_DIM))

    return out.reshape(batch, seq, d)
